# Initial kernel scaffold; baseline (speedup 1.0000x reference)
import jax
import jax.numpy as jnp
from jax.experimental import pallas as pl


def kernel(x, emb_weight):
    raise NotImplementedError("write your pallas kernel here")


# speedup vs baseline: 2.2993x; 2.2993x over previous
import functools

import jax
import jax.numpy as jnp
from jax import lax
from jax.experimental import pallas as pl
from jax.experimental.pallas import tpu as pltpu
from jax.experimental.pallas import tpu_sc as plsc

_VOTER_INPUT = 8
_SIGNAL_COUNT = 64
_N = 819200

_NC = 2
_NS = 16
_NW = _NC * _NS

_CHUNK = 1024
_IDXW = 128
_NGRP = _CHUNK // _IDXW
_B_PER_W = _N // _NW
_NCHUNK = _B_PER_W // _CHUNK


def _emb_body(x_hbm, table_hbm, out_hbm, xv, idx2d, rows_v, sem):
    wid = lax.axis_index("s") * _NC + lax.axis_index("c")
    wbase = wid * _B_PER_W

    lane8 = lax.iota(jnp.int32, 16) * _VOTER_INPUT

    def chunk_step(c, carry):
        row0 = pl.multiple_of(wbase + c * _CHUNK, _CHUNK)
        pltpu.sync_copy(
            x_hbm.at[pl.ds(row0 * _VOTER_INPUT, _CHUNK * _VOTER_INPUT)], xv
        )

        for k in range(_CHUNK // 16):
            acc = jnp.zeros((16,), jnp.int32)
            for j in range(_VOTER_INPUT):
                col = plsc.load_gather(
                    xv, [lane8 + (k * 16 * _VOTER_INPUT + j)]
                )
                acc = acc + col * (1 << (_VOTER_INPUT - 1 - j))
            g, o = divmod(k * 16, _IDXW)
            idx2d[g, pl.ds(o, 16)] = acc

        descs = [
            pltpu.async_copy(
                table_hbm.at[idx2d.at[g]],
                rows_v.at[pl.ds(g * _IDXW, _IDXW)],
                sem,
            )
            for g in range(_NGRP)
        ]
        for d in descs:
            d.wait()

        pltpu.sync_copy(rows_v, out_hbm.at[pl.ds(row0, _CHUNK)])
        return carry

    lax.fori_loop(0, _NCHUNK, chunk_step, 0)


@jax.jit
def _emb_lookup(x, emb_weight):
    mesh = plsc.VectorSubcoreMesh(core_axis_name="c", subcore_axis_name="s")
    run = functools.partial(
        pl.kernel,
        mesh=mesh,
        out_type=jax.ShapeDtypeStruct((_N, _SIGNAL_COUNT), jnp.float32),
        scratch_types=[
            pltpu.VMEM((_CHUNK * _VOTER_INPUT,), jnp.int32),
            pltpu.VMEM((_NGRP, _IDXW), jnp.int32),
            pltpu.VMEM((_CHUNK, _SIGNAL_COUNT), jnp.float32),
            pltpu.SemaphoreType.DMA,
        ],
        compiler_params=pltpu.CompilerParams(
            needs_layout_passes=False, use_tc_tiling_on_sc=False
        ),
    )(_emb_body)
    return run(x, emb_weight)


def kernel(x, emb_weight):
    x = x.reshape(_N * _VOTER_INPUT).astype(jnp.int32)
    out = _emb_lookup(x, emb_weight)
    return out.reshape(_N, 1, _SIGNAL_COUNT)
